# baseline (speedup 1.0000x reference)
import functools

import jax
import jax.numpy as jnp
from jax import lax
from jax.experimental import pallas as pl
from jax.experimental.pallas import tpu as pltpu
from jax.experimental.pallas import tpu_sc as plsc

B = 4096
N = 16384
HALF = N // 2
L = 16
NC = 2
NS = 16
NW = NC * NS
ROWS_PER_W = B // NW
KU = 8
CHUNK = L * KU
JBITS = 14
JMASK = (1 << JBITS) - 1
PCHUNK = 4096


@functools.cache
def _build(np2: int):
    mesh = plsc.VectorSubcoreMesh(
        core_axis_name="c", subcore_axis_name="s", num_cores=NC, num_subcores=NS
    )

    @functools.partial(
        pl.kernel,
        out_type=jax.ShapeDtypeStruct((B, N), jnp.float32),
        mesh=mesh,
        compiler_params=pltpu.CompilerParams(needs_layout_passes=False),
        scratch_types=[
            pltpu.VMEM((np2,), jnp.int32),
            pltpu.VMEM((np2,), jnp.float32),
            pltpu.VMEM((N,), jnp.float32),
            pltpu.VMEM((L,), jnp.int32),
            pltpu.VMEM((PCHUNK,), jnp.int32),
            pltpu.VMEM((HALF,), jnp.float32),
            pltpu.VMEM((HALF,), jnp.float32),
            pltpu.VMEM((HALF,), jnp.float32),
            pltpu.VMEM((HALF,), jnp.float32),
            pltpu.VMEM((N,), jnp.float32),
            pltpu.VMEM((N,), jnp.float32),
            pltpu.VMEM((N,), jnp.float32),
            pltpu.SemaphoreType.DMA,
            pltpu.SemaphoreType.DMA,
            pltpu.SemaphoreType.DMA,
            pltpu.SemaphoreType.DMA,
            pltpu.SemaphoreType.DMA,
            pltpu.SemaphoreType.DMA,
            pltpu.SemaphoreType.DMA,
        ],
    )
    def sc_kernel(inputs_hbm, w3_hbm, b_hbm, vel_hbm, packed_hbm, perm_hbm,
                  meta_hbm, out_hbm,
                  packed, vals, biasv, meta, pchunk,
                  xa0, xa1, xb0, xb1, y0, y1, y2,
                  sxa0, sxa1, sxb0, sxb1, sy0, sy1, sy2):
        wid = lax.axis_index("s") * NC + lax.axis_index("c")
        row0 = wid * ROWS_PER_W
        row_end = row0 + ROWS_PER_W
        ys = (y0, y1, y2)
        sys_ = (sy0, sy1, sy2)

        pltpu.sync_copy(packed_hbm, packed)
        pltpu.sync_copy(meta_hbm, meta)
        pltpu.sync_copy(w3_hbm, y0.at[pl.ds(0, np2)])
        pltpu.sync_copy(vel_hbm, y1)
        pltpu.sync_copy(b_hbm, biasv)

        @plsc.parallel_loop(0, np2 // L, unroll=KU)
        def val_body(t):
            o = t * L
            pk = packed[pl.ds(o, L)]
            jv = lax.bitwise_and(pk, JMASK)
            g = plsc.load_gather(y1, [jv])
            vals[pl.ds(o, L)] = y0[pl.ds(o, L)] * g

        for c in range(-(-np2 // PCHUNK)):
            cbase = c * PCHUNK
            cs = min(PCHUNK, np2 - cbase)
            pltpu.sync_copy(perm_hbm.at[pl.ds(cbase, cs)], pchunk.at[pl.ds(0, cs)])

            @plsc.parallel_loop(0, cs // L, unroll=KU)
            def perm_body(t):
                o = t * L
                pv = pchunk[pl.ds(o, L)]
                pk2 = plsc.load_gather(packed, [pv])
                vv2 = plsc.load_gather(vals, [pv])
                y2[pl.ds(cbase + o, L)] = plsc.bitcast(pk2, jnp.float32)
                y0[pl.ds(cbase + o, L)] = vv2

        @plsc.parallel_loop(0, np2 // L, unroll=KU)
        def copyback_body(t):
            o = t * L
            packed[pl.ds(o, L)] = plsc.bitcast(y2[pl.ds(o, L)], jnp.int32)
            vals[pl.ds(o, L)] = y0[pl.ds(o, L)]

        t_split = jnp.sum(meta[pl.ds(0, L)]) // L
        n_groups = np2 // L

        def bias_init(ybuf):
            @plsc.parallel_loop(0, N // L, unroll=KU)
            def bias_body(i):
                o = i * L
                ybuf[pl.ds(o, L)] = biasv[pl.ds(o, L)]

        def ab_loop(lo, hi, xr0, xr1, ya, yb, local_off):
            @plsc.parallel_loop(lo, hi, unroll=KU)
            def k_body(t):
                o = t * L
                pk = packed[pl.ds(o, L)]
                jv = lax.bitwise_and(pk, JMASK) - local_off
                iv = lax.shift_right_logical(pk, JBITS)
                vv = vals[pl.ds(o, L)]
                g0 = plsc.load_gather(xr0, [jv])
                g1 = plsc.load_gather(xr1, [jv])
                plsc.addupdate_scatter(ya, [iv], vv * g0)
                plsc.addupdate_scatter(yb, [iv], vv * g1)

        def do_pass(r0, ya, yb, sya, syb, wait_ya, wait_yb, prefetch):
            r1 = r0 + 1
            pltpu.async_copy(inputs_hbm.at[r0, pl.ds(HALF, HALF)], xb0, sxb0)
            pltpu.async_copy(inputs_hbm.at[r1, pl.ds(HALF, HALF)], xb1, sxb1)
            if wait_ya is not None:
                wait_ya()
            bias_init(ya)
            if wait_yb is not None:
                wait_yb()
            bias_init(yb)
            pltpu.make_async_copy(inputs_hbm.at[r0, pl.ds(0, HALF)], xa0, sxa0).wait()
            pltpu.make_async_copy(inputs_hbm.at[r1, pl.ds(0, HALF)], xa1, sxa1).wait()
            ab_loop(0, t_split, xa0, xa1, ya, yb, 0)
            if prefetch:
                @pl.when(r0 + 2 < row_end)
                def _():
                    pltpu.async_copy(inputs_hbm.at[r0 + 2, pl.ds(0, HALF)], xa0, sxa0)
                    pltpu.async_copy(inputs_hbm.at[r1 + 2, pl.ds(0, HALF)], xa1, sxa1)
            pltpu.make_async_copy(inputs_hbm.at[r0, pl.ds(HALF, HALF)], xb0, sxb0).wait()
            pltpu.make_async_copy(inputs_hbm.at[r1, pl.ds(HALF, HALF)], xb1, sxb1).wait()
            ab_loop(t_split, n_groups, xb0, xb1, ya, yb, HALF)
            pltpu.async_copy(ya, out_hbm.at[r0], sya)
            pltpu.async_copy(yb, out_hbm.at[r1], syb)

        pltpu.async_copy(inputs_hbm.at[row0, pl.ds(0, HALF)], xa0, sxa0)
        pltpu.async_copy(inputs_hbm.at[row0 + 1, pl.ds(0, HALF)], xa1, sxa1)

        def wait_store(m, r):
            def w():
                pltpu.make_async_copy(ys[m], out_hbm.at[r], sys_[m]).wait()
            return w

        def guarded(q, m, r):
            def w():
                @pl.when(q > 0)
                def _():
                    pltpu.make_async_copy(ys[m], out_hbm.at[r], sys_[m]).wait()
            return w

        def superpass(q, c):
            base = row0 + 6 * q
            do_pass(base, y0, y1, sy0, sy1,
                    guarded(q, 0, base - 3), guarded(q, 1, base - 2),
                    True)
            do_pass(base + 2, y2, y0, sy2, sy0,
                    guarded(q, 2, base - 1), wait_store(0, base),
                    True)
            do_pass(base + 4, y1, y2, sy1, sy2,
                    wait_store(1, base + 1), wait_store(2, base + 2),
                    True)
            return c

        nq = (ROWS_PER_W // 2 - 1) // 3
        lax.fori_loop(0, nq, superpass, 0)

        fr = row0 + 6 * nq
        do_pass(fr, y0, y1, sy0, sy1,
                wait_store(0, fr - 3), wait_store(1, fr - 2),
                False)

        pltpu.make_async_copy(y0, out_hbm.at[fr], sy0).wait()
        pltpu.make_async_copy(y1, out_hbm.at[fr + 1], sy1).wait()
        pltpu.make_async_copy(y2, out_hbm.at[fr - 1], sy2).wait()

    return sc_kernel


def kernel(inputs, W3, b, velocity, I, J):
    nnz = W3.shape[0]
    np2 = ((nnz + CHUNK - 1) // CHUNK) * CHUNK
    npad = np2 - nnz
    I32 = I.astype(jnp.int32)
    J32 = J.astype(jnp.int32)
    in_b = J32 >= HALF

    ar = jnp.arange(nnz, dtype=jnp.int32)
    first = jnp.concatenate([jnp.ones((1,), jnp.bool_), I32[1:] != I32[:-1]])
    seg_base = lax.cummax(jnp.where(first, ar, 0))
    rank = ar - seg_base

    key_real = jnp.left_shift(in_b.astype(jnp.int32), 15) | rank

    a_count = jnp.sum(~in_b).astype(jnp.int32)
    g = (-a_count) % L
    pidx = jnp.arange(npad, dtype=jnp.int32)
    is_gap_pad = pidx < g
    key_pad = jnp.where(is_gap_pad,
                        0x4000 + pidx,
                        jnp.left_shift(1, 15) | (0x4000 + pidx))
    pad_j = jnp.where(is_gap_pad, 0, HALF)
    pad_packed = jnp.left_shift(pidx, JBITS) | pad_j

    packed = jnp.concatenate([jnp.left_shift(I32, JBITS) | J32, pad_packed])
    W3p = jnp.concatenate([W3, jnp.zeros((npad,), jnp.float32)])
    key = jnp.concatenate([key_real, key_pad])
    perm = jnp.argsort(key, stable=True).astype(jnp.int32)
    meta = jnp.zeros((L,), jnp.int32).at[0].set(a_count + g)
    return _build(np2)(inputs, W3p, b, velocity, packed, perm, meta)
